# baseline (speedup 1.0000x reference)
import functools

import jax
import jax.numpy as jnp
from jax import lax
from jax.experimental import pallas as pl
from jax.experimental.pallas import tpu as pltpu
from jax.experimental.pallas import tpu_sc as plsc

VOCAB = 100000
D_MODEL = 64
BATCH = 4096
HIST = 200
B_TOTAL = BATCH * HIST

_INFO = plsc.get_sparse_core_info()
_NC = _INFO.num_cores
_NS = _INFO.num_subcores
_L = _INFO.num_lanes
_NW = _NC * _NS
_BT = BATCH // _NW
_B_PER_W = _BT * HIST
_DT = D_MODEL // 8


def _emb_body(tok_hbm, w_hbm, out_hbm, tok_v, tokT_v, gbuf, tbuf, gsem, osem):
  wid = lax.axis_index("s") * _NC + lax.axis_index("c")
  pltpu.sync_copy(tok_hbm.at[pl.ds(wid * _B_PER_W, _B_PER_W)], tok_v)

  lane = lax.iota(jnp.int32, _L)

  @pl.loop(0, HIST)
  def _tok_t(h):
    for j in range(_BT // _L):
      idx = (lane + j * _L) * HIST + h
      tokT_v[h, pl.ds(j * _L, _L)] = plsc.load_gather(tok_v, [idx])

  def gather(h, s):
    return pltpu.make_async_copy(
        w_hbm.at[tokT_v.at[h]], gbuf.at[s], gsem.at[s])

  def store(h, s):
    return pltpu.make_async_copy(
        tbuf.at[s], out_hbm.at[h, :, wid], osem.at[s])

  gather(0, 0).start()
  gather(1, 1).start()

  @pl.loop(0, HIST)
  def _h(h):
    s = lax.rem(h, 2)
    gather(h, s).wait()

    @pl.when(h >= 2)
    def _drain():
      store(h - 2, s).wait()

    gsrc = gbuf.at[s]
    tdst = tbuf.at[s]
    for dt in range(_DT):
      for di in range(8):
        d = dt * 8 + di
        dvec = jnp.full((_L,), d, jnp.int32)
        for j in range(_BT // _L):
          tdst[dt, di, pl.ds(j * _L, _L)] = plsc.load_gather(
              gsrc, [lane + j * _L, dvec])

    store(h, s).start()

    @pl.when(h + 2 < HIST)
    def _next():
      gather(h + 2, s).start()

  store(HIST - 2, 0).wait()
  store(HIST - 1, 1).wait()


_emb = functools.partial(
    pl.kernel,
    out_type=jax.ShapeDtypeStruct((HIST, _DT, _NW, 8, 128), jnp.float32),
    mesh=plsc.VectorSubcoreMesh(core_axis_name="c", subcore_axis_name="s"),
    scratch_types=[
        pltpu.VMEM((_B_PER_W,), jnp.int32),
        pltpu.VMEM((HIST, _BT), jnp.int32),
        pltpu.VMEM((2, _BT, D_MODEL), jnp.float32),
        pltpu.VMEM((2, _DT, 8, 128), jnp.float32),
        pltpu.SemaphoreType.DMA((2,)),
        pltpu.SemaphoreType.DMA((2,)),
    ],
    compiler_params=pltpu.CompilerParams(
        use_tc_tiling_on_sc=False, needs_layout_passes=False),
)(_emb_body)


@jax.jit
def kernel(token_ids, weight):
  tok = token_ids.reshape(B_TOTAL).astype(jnp.int32)
  out5 = _emb(tok, weight)
  return out5.transpose(2, 4, 0, 1, 3).reshape(BATCH, HIST, D_MODEL)


# speedup vs baseline: 1.2912x; 1.2912x over previous
import functools

import jax
import jax.numpy as jnp
from jax import lax
from jax.experimental import pallas as pl
from jax.experimental.pallas import tpu as pltpu
from jax.experimental.pallas import tpu_sc as plsc

VOCAB = 100000
D_MODEL = 64
BATCH = 4096
HIST = 200
B_TOTAL = BATCH * HIST

_INFO = plsc.get_sparse_core_info()
_NC = _INFO.num_cores
_NS = _INFO.num_subcores
_L = _INFO.num_lanes
_NW = _NC * _NS
_BT = BATCH // _NW
_B_PER_W = _BT * HIST
_DT = D_MODEL // 8


def _emb_body(tok_hbm, w_hbm, out_hbm, tok_v, tokT_v, gbuf, tbuf, gsem, osem):
  wid = lax.axis_index("s") * _NC + lax.axis_index("c")
  pltpu.sync_copy(tok_hbm.at[pl.ds(wid * _B_PER_W, _B_PER_W)], tok_v)

  lane = lax.iota(jnp.int32, _L)
  rows = [lane + j * _L for j in range(_BT // _L)]

  @pl.loop(0, HIST)
  def _tok_t(h):
    vals = [plsc.load_gather(tok_v, [r * HIST + h]) for r in rows]
    for j, v in enumerate(vals):
      tokT_v[h, pl.ds(j * _L, _L)] = v

  def gather(h, s):
    return pltpu.make_async_copy(
        w_hbm.at[tokT_v.at[h]], gbuf.at[s], gsem.at[s])

  def store(h, s):
    return pltpu.make_async_copy(
        tbuf.at[s], out_hbm.at[h, :, wid], osem.at[s])

  gather(0, 0).start()
  gather(1, 1).start()

  @pl.loop(0, HIST)
  def _h(h):
    s = lax.rem(h, 2)
    gather(h, s).wait()

    @pl.when(h >= 2)
    def _drain():
      store(h - 2, s).wait()

    gsrc = gbuf.at[s]
    tdst = tbuf.at[s]
    for dt in range(_DT):
      for di in range(0, 8, 2):
        vals = []
        for dd in range(2):
          dvec = jnp.full((_L,), dt * 8 + di + dd, jnp.int32)
          vals.extend(plsc.load_gather(gsrc, [r, dvec]) for r in rows)
        for k, v in enumerate(vals):
          tdst[dt, di + k // 8, pl.ds((k % 8) * _L, _L)] = v

    store(h, s).start()

    @pl.when(h + 2 < HIST)
    def _next():
      gather(h + 2, s).start()

  store(HIST - 2, 0).wait()
  store(HIST - 1, 1).wait()


_emb = functools.partial(
    pl.kernel,
    out_type=jax.ShapeDtypeStruct((HIST, _DT, _NW, 8, 128), jnp.float32),
    mesh=plsc.VectorSubcoreMesh(core_axis_name="c", subcore_axis_name="s"),
    scratch_types=[
        pltpu.VMEM((_B_PER_W,), jnp.int32),
        pltpu.VMEM((HIST, _BT), jnp.int32),
        pltpu.VMEM((2, _BT, D_MODEL), jnp.float32),
        pltpu.VMEM((2, _DT, 8, 128), jnp.float32),
        pltpu.SemaphoreType.DMA((2,)),
        pltpu.SemaphoreType.DMA((2,)),
    ],
    compiler_params=pltpu.CompilerParams(
        use_tc_tiling_on_sc=False, needs_layout_passes=False),
)(_emb_body)


@jax.jit
def kernel(token_ids, weight):
  tok = token_ids.reshape(B_TOTAL).astype(jnp.int32)
  out5 = _emb(tok, weight)
  return out5.transpose(2, 4, 0, 1, 3).reshape(BATCH, HIST, D_MODEL)


# speedup vs baseline: 3.3716x; 2.6111x over previous
import functools

import jax
import jax.numpy as jnp
from jax import lax
from jax.experimental import pallas as pl
from jax.experimental.pallas import tpu as pltpu
from jax.experimental.pallas import tpu_sc as plsc

VOCAB = 100000
D_MODEL = 64
D_PAD = 65
BATCH = 4096
HIST = 200
B_TOTAL = BATCH * HIST

_INFO = plsc.get_sparse_core_info()
_NC = _INFO.num_cores
_NS = _INFO.num_subcores
_L = _INFO.num_lanes
_NW = _NC * _NS
_BT = BATCH // _NW
_B_PER_W = _BT * HIST
_DT = D_MODEL // 8


def _emb_body(tok_hbm, w_hbm, out_hbm, tok_v, tokT_v, gbuf, tbuf, gsem, osem):
  wid = lax.axis_index("s") * _NC + lax.axis_index("c")
  pltpu.sync_copy(tok_hbm.at[pl.ds(wid * _B_PER_W, _B_PER_W)], tok_v)

  lane = lax.iota(jnp.int32, _L)
  rows = [lane + j * _L for j in range(_BT // _L)]
  zcol = lane * 0

  @pl.loop(0, HIST)
  def _tok_t(h):
    vals = [plsc.load_gather(tok_v, [r * HIST + h]) for r in rows]
    for j, v in enumerate(vals):
      tokT_v[h, pl.ds(j * _L, _L)] = v

  def gather(h, s):
    return pltpu.make_async_copy(
        w_hbm.at[tokT_v.at[h]], gbuf.at[s], gsem.at[s])

  def store(h, s):
    return pltpu.make_async_copy(
        tbuf.at[s], out_hbm.at[h, :, wid], osem.at[s])

  gather(0, 0).start()
  gather(1, 1).start()

  @pl.loop(0, HIST)
  def _h(h):
    s = lax.rem(h, 2)
    gather(h, s).wait()

    @pl.when(h >= 2)
    def _drain():
      store(h - 2, s).wait()

    gsrc = gbuf.at[s]
    tdst = tbuf.at[s]
    for dt in range(_DT):
      for di in range(0, 8, 2):
        vals = []
        for dd in range(2):
          dvec = zcol + (dt * 8 + di + dd)
          vals.extend(plsc.load_gather(gsrc, [r, dvec]) for r in rows)
        for k, v in enumerate(vals):
          tdst[dt, di + k // 8, pl.ds((k % 8) * _L, _L)] = v

    store(h, s).start()

    @pl.when(h + 2 < HIST)
    def _next():
      gather(h + 2, s).start()

  store(HIST - 2, 0).wait()
  store(HIST - 1, 1).wait()


_emb = functools.partial(
    pl.kernel,
    out_type=jax.ShapeDtypeStruct((HIST, _DT, _NW, 8, 128), jnp.float32),
    mesh=plsc.VectorSubcoreMesh(core_axis_name="c", subcore_axis_name="s"),
    scratch_types=[
        pltpu.VMEM((_B_PER_W,), jnp.int32),
        pltpu.VMEM((HIST, _BT), jnp.int32),
        pltpu.VMEM((2, _BT, D_PAD), jnp.float32),
        pltpu.VMEM((2, _DT, 8, 128), jnp.float32),
        pltpu.SemaphoreType.DMA((2,)),
        pltpu.SemaphoreType.DMA((2,)),
    ],
    compiler_params=pltpu.CompilerParams(
        use_tc_tiling_on_sc=False, needs_layout_passes=False),
)(_emb_body)


@jax.jit
def kernel(token_ids, weight):
  tok = token_ids.reshape(B_TOTAL).astype(jnp.int32)
  wpad = jnp.pad(weight, ((0, 0), (0, D_PAD - D_MODEL)))
  out5 = _emb(tok, wpad)
  return out5.transpose(2, 4, 0, 1, 3).reshape(BATCH, HIST, D_MODEL)
